# baseline (speedup 1.0000x reference)
import functools

import jax
import jax.numpy as jnp
from jax import lax
from jax.experimental import pallas as pl
from jax.experimental.pallas import tpu as pltpu
from jax.experimental.pallas import tpu_sc as plsc

RES_X = 1024
RES_Y = 1024
D_HALF = 256
L = 16
NC, NS = 2, 16
NW = NC * NS
B = 16 * 32 * 128
TPW = B // NW
RPW = 4 * TPW
CHUNK_ROWS = 64
NCHUNK = RPW // CHUNK_ROWS
NBUF = 8
LA = NBUF // 2

Y_OFF = RES_X + 8
ZERO_X = RES_X
ZERO_Y = Y_OFF + RES_Y

_mesh = plsc.VectorSubcoreMesh(core_axis_name="c", subcore_axis_name="s")


@functools.partial(
    pl.kernel,
    out_type=jax.ShapeDtypeStruct((4 * B, 128), jnp.float32),
    mesh=_mesh,
    scratch_types=[
        pltpu.VMEM((TPW // 128, 128), jnp.float32),
        pltpu.VMEM((TPW // 128, 128), jnp.float32),
        pltpu.VMEM((2 * L,), jnp.float32),
        pltpu.VMEM((RPW,), jnp.int32),
        [pltpu.VMEM((CHUNK_ROWS, 128), jnp.float32)] * NBUF,
        pltpu.VMEM_SHARED((4 * 1032, 128), jnp.float32),
        [pltpu.SemaphoreType.DMA] * NBUF,
        [pltpu.SemaphoreType.DMA] * NBUF,
        pltpu.SemaphoreType.DMA,
    ],
)
def _pos_lookup(coord_hbm, size_hbm, table_hbm, out_hbm,
                cx_v, cy_v, size_v, idx_v, rows, table_sh, gsem, wsem, tsem):
    sid = lax.axis_index("s")
    wid = sid * NC + lax.axis_index("c")

    @pl.when(sid == 0)
    def _():
        pltpu.async_copy(table_hbm, table_sh, tsem)

    r0 = wid * 16
    pltpu.sync_copy(coord_hbm.at[0, pl.ds(r0, 16)], cx_v)
    pltpu.sync_copy(coord_hbm.at[1, pl.ds(r0, 16)], cy_v)
    pltpu.sync_copy(size_hbm, size_v)

    sw_vec = size_v[pl.ds(0, L)]
    sh_vec = size_v[pl.ds(L, L)]
    iota = lax.iota(jnp.int32, L)
    i8lo = iota & 7
    i8hi = i8lo + 8
    bitv = (iota >> 3) & 1
    hi_bit = bitv << 3
    zx_vec = jnp.full((L,), ZERO_X, jnp.int32)
    zy_vec = jnp.full((L,), ZERO_Y, jnp.int32)
    dnums = lax.GatherDimensionNumbers(
        offset_dims=(), collapsed_slice_dims=(0,), start_index_map=(0,))

    def perm(v, idx):
        return lax.gather(v, idx[:, None], dnums, (1,),
                          mode=lax.GatherScatterMode.PROMISE_IN_BOUNDS)

    def phys(t):
        return ((t >> 3) << 4) + hi_bit + (t & 7)

    def compute_idx(c):
        rr = c >> 3
        m = c & 7
        xv = cx_v[rr, pl.ds(m * L, L)]
        yv = cy_v[rr, pl.ds(m * L, L)]
        msk = xv < 0.0
        rx = jnp.clip((jnp.float32(RES_X) * xv / sw_vec).astype(jnp.int32),
                      0, RES_X - 1)
        ry = jnp.clip((jnp.float32(RES_Y) * yv / sh_vec).astype(jnp.int32),
                      0, RES_Y - 1) + Y_OFF
        rx = jnp.where(msk, zx_vec, rx)
        ry = jnp.where(msk, zy_vec, ry)
        base = c * 4 * L
        idx_v[pl.ds(base, L)] = phys(perm(rx, i8lo))
        idx_v[pl.ds(base + L, L)] = phys(perm(ry, i8lo))
        idx_v[pl.ds(base + 2 * L, L)] = phys(perm(rx, i8hi))
        idx_v[pl.ds(base + 3 * L, L)] = phys(perm(ry, i8hi))

    def _prologue(c, _):
        compute_idx(c)
        return 0

    lax.fori_loop(0, LA, _prologue, 0)

    @pl.when(sid == 0)
    def _():
        pltpu.make_async_copy(table_hbm, table_sh, tsem).wait()

    plsc.subcore_barrier()

    obase = RPW * wid

    def idx_slice(k):
        return idx_v.at[pl.ds(k * CHUNK_ROWS, CHUNK_ROWS)]

    def out_slice(k):
        return out_hbm.at[pl.ds(obase + k * CHUNK_ROWS, CHUNK_ROWS)]

    def fire_gather(k, b):
        pltpu.async_copy(table_sh.at[idx_slice(k)], rows[b], gsem[b])

    def wait_gather(b):
        pltpu.make_async_copy(table_sh.at[idx_slice(0)], rows[b],
                              gsem[b]).wait()

    def wait_write(b):
        pltpu.make_async_copy(rows[b], out_slice(0), wsem[b]).wait()

    for b in range(LA):
        fire_gather(b, b)

    def ring_body(kk, _):
        for b in range(NBUF):
            k = kk * NBUF + b
            if b < LA:
                compute_idx(k + LA)
            else:
                @pl.when(kk + 1 < NCHUNK // NBUF)
                def _():
                    compute_idx(k + LA)

            wait_gather(b)
            pltpu.async_copy(rows[b], out_slice(k), wsem[b])
            bw = (b + LA) % NBUF
            if b < LA:
                @pl.when(kk >= 1)
                def _():
                    wait_write(bw)

                fire_gather(k + LA, bw)
            else:
                wait_write(bw)

                @pl.when(kk + 1 < NCHUNK // NBUF)
                def _():
                    fire_gather(k + LA, bw)
        return 0

    lax.fori_loop(0, NCHUNK // NBUF, ring_body, 0)
    for b in range(NBUF - LA, NBUF):
        wait_write(b)


def kernel(coordinate, size, x_embedding, y_embedding):
    coord_t = jnp.moveaxis(coordinate, 3, 0).reshape(2, B // 128, 128)
    z8 = jnp.zeros((8, D_HALF), jnp.float32)
    t = jnp.concatenate([x_embedding, z8, y_embedding, z8])
    table = t.reshape(258, 8, 2, 128).swapaxes(1, 2).reshape(4128, 128)
    sizes = jnp.concatenate([jnp.broadcast_to(size[1], (L,)),
                             jnp.broadcast_to(size[0], (L,))])
    out = _pos_lookup(coord_t, sizes, table)
    out = out.reshape(16, 32, 16, 4, 8, 128).swapaxes(3, 4)
    return out.reshape(16, 32, 128, 2 * D_HALF)


# speedup vs baseline: 1.0235x; 1.0235x over previous
import functools

import jax
import jax.numpy as jnp
from jax import lax
from jax.experimental import pallas as pl
from jax.experimental.pallas import tpu as pltpu
from jax.experimental.pallas import tpu_sc as plsc

RES_X = 1024
RES_Y = 1024
D_HALF = 256
L = 16
NC, NS = 2, 16
NW = NC * NS
B = 16 * 32 * 128
TPW = B // NW
RPW = 4 * TPW
CHUNK_ROWS = 64
NCHUNK = RPW // CHUNK_ROWS
NBUF = 8
LA = NBUF // 2

Y_OFF = RES_X + 8
ZERO_X = RES_X
ZERO_Y = Y_OFF + RES_Y

_mesh = plsc.VectorSubcoreMesh(core_axis_name="c", subcore_axis_name="s")


@functools.partial(
    pl.kernel,
    out_type=jax.ShapeDtypeStruct((4 * B, 128), jnp.float32),
    mesh=_mesh,
    scratch_types=[
        pltpu.VMEM((TPW // 128, 128), jnp.float32),
        pltpu.VMEM((TPW // 128, 128), jnp.float32),
        pltpu.VMEM((L,), jnp.float32),
        pltpu.VMEM((RPW,), jnp.int32),
        [pltpu.VMEM((CHUNK_ROWS, 128), jnp.float32)] * NBUF,
        pltpu.VMEM_SHARED((4 * 1032, 128), jnp.float32),
        [pltpu.SemaphoreType.DMA] * NBUF,
        [pltpu.SemaphoreType.DMA] * NBUF,
        pltpu.SemaphoreType.DMA,
    ],
)
def _pos_lookup(coord_hbm, size_hbm, table_hbm, out_hbm,
                cx_v, cy_v, size_v, idx_v, rows, table_sh, gsem, wsem, tsem):
    sid = lax.axis_index("s")
    wid = sid * NC + lax.axis_index("c")

    @pl.when(sid == 0)
    def _():
        pltpu.async_copy(table_hbm, table_sh, tsem)

    r0 = wid * 16
    pltpu.sync_copy(coord_hbm.at[0, pl.ds(r0, 16)], cx_v)
    pltpu.sync_copy(coord_hbm.at[1, pl.ds(r0, 16)], cy_v)
    pltpu.sync_copy(size_hbm, size_v)

    iota = lax.iota(jnp.int32, L)
    i8lo = iota & 7
    i8hi = i8lo + 8
    bitv = (iota >> 3) & 1
    hi_bit = bitv << 3
    zx_vec = jnp.full((L,), ZERO_X, jnp.int32)
    zy_vec = jnp.full((L,), ZERO_Y, jnp.int32)
    dnums = lax.GatherDimensionNumbers(
        offset_dims=(), collapsed_slice_dims=(0,), start_index_map=(0,))

    def perm(v, idx):
        return lax.gather(v, idx[:, None], dnums, (1,),
                          mode=lax.GatherScatterMode.PROMISE_IN_BOUNDS)

    sv = size_v[pl.ds(0, L)]
    sw_vec = perm(sv, iota & 0 | 1)
    sh_vec = perm(sv, iota & 0)

    def phys(t):
        return ((t >> 3) << 4) + hi_bit + (t & 7)

    def compute_idx(c):
        rr = c >> 3
        m = c & 7
        xv = cx_v[rr, pl.ds(m * L, L)]
        yv = cy_v[rr, pl.ds(m * L, L)]
        msk = xv < 0.0
        rx = jnp.clip((jnp.float32(RES_X) * xv / sw_vec).astype(jnp.int32),
                      0, RES_X - 1)
        ry = jnp.clip((jnp.float32(RES_Y) * yv / sh_vec).astype(jnp.int32),
                      0, RES_Y - 1) + Y_OFF
        rx = jnp.where(msk, zx_vec, rx)
        ry = jnp.where(msk, zy_vec, ry)
        base = c * 4 * L
        idx_v[pl.ds(base, L)] = phys(perm(rx, i8lo))
        idx_v[pl.ds(base + L, L)] = phys(perm(ry, i8lo))
        idx_v[pl.ds(base + 2 * L, L)] = phys(perm(rx, i8hi))
        idx_v[pl.ds(base + 3 * L, L)] = phys(perm(ry, i8hi))

    def _prologue(c, _):
        compute_idx(c)
        return 0

    lax.fori_loop(0, LA, _prologue, 0)

    @pl.when(sid == 0)
    def _():
        pltpu.make_async_copy(table_hbm, table_sh, tsem).wait()

    plsc.subcore_barrier()

    obase = RPW * wid

    def idx_slice(k):
        return idx_v.at[pl.ds(k * CHUNK_ROWS, CHUNK_ROWS)]

    def out_slice(k):
        return out_hbm.at[pl.ds(obase + k * CHUNK_ROWS, CHUNK_ROWS)]

    def fire_gather(k, b):
        pltpu.async_copy(table_sh.at[idx_slice(k)], rows[b], gsem[b])

    def wait_gather(b):
        pltpu.make_async_copy(table_sh.at[idx_slice(0)], rows[b],
                              gsem[b]).wait()

    def wait_write(b):
        pltpu.make_async_copy(rows[b], out_slice(0), wsem[b]).wait()

    for b in range(LA):
        fire_gather(b, b)

    def ring_body(kk, _):
        for b in range(NBUF):
            k = kk * NBUF + b
            if b < LA:
                compute_idx(k + LA)
            else:
                @pl.when(kk + 1 < NCHUNK // NBUF)
                def _():
                    compute_idx(k + LA)

            wait_gather(b)
            pltpu.async_copy(rows[b], out_slice(k), wsem[b])
            bw = (b + LA) % NBUF
            if b < LA:
                @pl.when(kk >= 1)
                def _():
                    wait_write(bw)

                fire_gather(k + LA, bw)
            else:
                wait_write(bw)

                @pl.when(kk + 1 < NCHUNK // NBUF)
                def _():
                    fire_gather(k + LA, bw)
        return 0

    lax.fori_loop(0, NCHUNK // NBUF, ring_body, 0)
    for b in range(NBUF - LA, NBUF):
        wait_write(b)


def kernel(coordinate, size, x_embedding, y_embedding):
    coord_t = jnp.moveaxis(coordinate, 3, 0).reshape(2, B // 128, 128)
    z8 = jnp.zeros((8, D_HALF), jnp.float32)
    t = jnp.concatenate([x_embedding, z8, y_embedding, z8])
    table = t.reshape(258, 8, 2, 128).swapaxes(1, 2).reshape(4128, 128)
    sizes = jnp.broadcast_to(size, (L // 2, 2)).reshape(L)
    out = _pos_lookup(coord_t, sizes, table)
    out = out.reshape(16, 32, 16, 4, 8, 128).swapaxes(3, 4)
    return out.reshape(16, 32, 128, 2 * D_HALF)


# speedup vs baseline: 1.0243x; 1.0008x over previous
import functools

import jax
import jax.numpy as jnp
from jax import lax
from jax.experimental import pallas as pl
from jax.experimental.pallas import tpu as pltpu
from jax.experimental.pallas import tpu_sc as plsc

RES_X = 1024
RES_Y = 1024
D_HALF = 256
L = 16
NC, NS = 2, 16
NW = NC * NS
B = 16 * 32 * 128
TPW = B // NW
RPW = 4 * TPW
CHUNK_ROWS = 64
NCHUNK = RPW // CHUNK_ROWS
NBUF = 8
LA = NBUF // 2

Y_OFF = RES_X + 8
ZERO_X = RES_X
ZERO_Y = Y_OFF + RES_Y

_mesh = plsc.VectorSubcoreMesh(core_axis_name="c", subcore_axis_name="s")


@functools.partial(
    pl.kernel,
    out_type=jax.ShapeDtypeStruct((4 * B, 128), jnp.float32),
    mesh=_mesh,
    scratch_types=[
        pltpu.VMEM((TPW // 128, 128), jnp.float32),
        pltpu.VMEM((TPW // 128, 128), jnp.float32),
        pltpu.VMEM((L,), jnp.float32),
        pltpu.VMEM((RPW,), jnp.int32),
        [pltpu.VMEM((CHUNK_ROWS, 128), jnp.float32)] * NBUF,
        pltpu.VMEM_SHARED((4 * 1032, 128), jnp.float32),
        [pltpu.SemaphoreType.DMA] * NBUF,
        [pltpu.SemaphoreType.DMA] * NBUF,
        pltpu.SemaphoreType.DMA,
    ],
)
def _pos_lookup(coord_hbm, size_hbm, table_hbm, out_hbm,
                cx_v, cy_v, size_v, idx_v, rows, table_sh, gsem, wsem, tsem):
    sid = lax.axis_index("s")
    wid = sid * NC + lax.axis_index("c")

    @pl.when(sid == 0)
    def _():
        pltpu.async_copy(table_hbm, table_sh, tsem)

    r0 = wid * 16
    pltpu.sync_copy(coord_hbm.at[0, pl.ds(r0, 16)], cx_v)
    pltpu.sync_copy(coord_hbm.at[1, pl.ds(r0, 16)], cy_v)
    pltpu.sync_copy(size_hbm, size_v)

    iota = lax.iota(jnp.int32, L)
    i8lo = iota & 7
    i8hi = i8lo + 8
    bitv = (iota >> 3) & 1
    hi_bit = bitv << 3
    zx_vec = jnp.full((L,), ZERO_X, jnp.int32)
    zy_vec = jnp.full((L,), ZERO_Y, jnp.int32)
    dnums = lax.GatherDimensionNumbers(
        offset_dims=(), collapsed_slice_dims=(0,), start_index_map=(0,))

    def perm(v, idx):
        return lax.gather(v, idx[:, None], dnums, (1,),
                          mode=lax.GatherScatterMode.PROMISE_IN_BOUNDS)

    sv = size_v[pl.ds(0, L)]
    sw_vec = perm(sv, jnp.ones((L,), jnp.int32))
    sh_vec = perm(sv, jnp.zeros((L,), jnp.int32))

    def phys(t):
        return ((t >> 3) << 4) + hi_bit + (t & 7)

    def compute_idx(c):
        rr = c >> 3
        m = c & 7
        xv = cx_v[rr, pl.ds(m * L, L)]
        yv = cy_v[rr, pl.ds(m * L, L)]
        msk = xv < 0.0
        rx = jnp.clip((jnp.float32(RES_X) * xv / sw_vec).astype(jnp.int32),
                      0, RES_X - 1)
        ry = jnp.clip((jnp.float32(RES_Y) * yv / sh_vec).astype(jnp.int32),
                      0, RES_Y - 1) + Y_OFF
        rx = jnp.where(msk, zx_vec, rx)
        ry = jnp.where(msk, zy_vec, ry)
        base = c * 4 * L
        idx_v[pl.ds(base, L)] = phys(perm(rx, i8lo))
        idx_v[pl.ds(base + L, L)] = phys(perm(ry, i8lo))
        idx_v[pl.ds(base + 2 * L, L)] = phys(perm(rx, i8hi))
        idx_v[pl.ds(base + 3 * L, L)] = phys(perm(ry, i8hi))

    def _prologue(c, _):
        compute_idx(c)
        return 0

    lax.fori_loop(0, LA, _prologue, 0)

    @pl.when(sid == 0)
    def _():
        pltpu.make_async_copy(table_hbm, table_sh, tsem).wait()

    plsc.subcore_barrier()

    obase = RPW * wid

    def idx_slice(k):
        return idx_v.at[pl.ds(k * CHUNK_ROWS, CHUNK_ROWS)]

    def out_slice(k):
        return out_hbm.at[pl.ds(obase + k * CHUNK_ROWS, CHUNK_ROWS)]

    def fire_gather(k, b):
        pltpu.async_copy(table_sh.at[idx_slice(k)], rows[b], gsem[b])

    def wait_gather(b):
        pltpu.make_async_copy(table_sh.at[idx_slice(0)], rows[b],
                              gsem[b]).wait()

    def wait_write(b):
        pltpu.make_async_copy(rows[b], out_slice(0), wsem[b]).wait()

    for b in range(LA):
        fire_gather(b, b)

    def ring_body(kk, _):
        for b in range(NBUF):
            k = kk * NBUF + b
            if b < LA:
                compute_idx(k + LA)
            else:
                @pl.when(kk + 1 < NCHUNK // NBUF)
                def _():
                    compute_idx(k + LA)

            wait_gather(b)
            pltpu.async_copy(rows[b], out_slice(k), wsem[b])
            bw = (b + LA) % NBUF
            if b < LA:
                @pl.when(kk >= 1)
                def _():
                    wait_write(bw)

                fire_gather(k + LA, bw)
            else:
                wait_write(bw)

                @pl.when(kk + 1 < NCHUNK // NBUF)
                def _():
                    fire_gather(k + LA, bw)
        return 0

    lax.fori_loop(0, NCHUNK // NBUF, ring_body, 0)
    for b in range(NBUF - LA, NBUF):
        wait_write(b)


def kernel(coordinate, size, x_embedding, y_embedding):
    coord_t = jnp.moveaxis(coordinate, 3, 0).reshape(2, B // 128, 128)
    z8 = jnp.zeros((8, D_HALF), jnp.float32)
    t = jnp.concatenate([x_embedding, z8, y_embedding, z8])
    table = t.reshape(258, 8, 2, 128).swapaxes(1, 2).reshape(4128, 128)
    sizes = jnp.broadcast_to(size, (L // 2, 2)).reshape(L)
    out = _pos_lookup(coord_t, sizes, table)
    out = out.reshape(16, 32, 16, 4, 8, 128).swapaxes(3, 4)
    return out.reshape(16, 32, 128, 2 * D_HALF)
